# baseline (speedup 1.0000x reference)
import jax
import jax.numpy as jnp
from jax import lax
from jax.experimental import pallas as pl
from jax.experimental.pallas import tpu as pltpu


def _vq_tile(f_ref, c_ref, ids_ref, sum_ref, csq_ref, c16_ref, iota_ref):
    i = pl.program_id(0)

    @pl.when(i == 0)
    def _():
        c = c_ref[...]
        csq_ref[...] = jnp.sum(c * c, axis=1)
        c16_ref[...] = c.astype(jnp.bfloat16)
        iota_ref[...] = lax.broadcasted_iota(
            jnp.int32, iota_ref.shape, 1).astype(jnp.float32)

    f = f_ref[...]
    TN = f.shape[0]
    C = csq_ref.shape[0]
    CK = 1024
    fsq = jnp.sum(f * f, axis=1, keepdims=True)
    f16x2 = f.astype(jnp.bfloat16) * jnp.bfloat16(2.0)

    def body(k, carry):
        m_run, idx_run = carry
        b2 = lax.dot_general(
            f16x2, c16_ref[pl.ds(k * CK, CK), :],
            dimension_numbers=(((1,), (1,)), ((), ())),
            preferred_element_type=jnp.float32,
        )
        d = (fsq - b2) + csq_ref[pl.ds(k * CK, CK)][None, :]
        mc = jnp.min(d, axis=1, keepdims=True)
        idxc = jnp.min(
            jnp.where(d == mc, iota_ref[:, pl.ds(k * CK, CK)],
                      jnp.float32(3.0e38)), axis=1)
        upd = mc < m_run
        return (jnp.where(upd, mc, m_run),
                jnp.where(upd[:, 0], idxc, idx_run))

    m0 = jnp.full((TN, 1), jnp.float32(jnp.inf))
    i0 = jnp.zeros((TN,), jnp.float32)
    m, idx = lax.fori_loop(0, C // CK, body, (m0, i0))
    ids_ref[0, 0, :] = idx.astype(jnp.int32)
    sum_ref[0, 0, :] = jnp.full((128,), jnp.sum(m), dtype=jnp.float32)


def kernel(features, codebook):
    B, T, D = features.shape
    C = codebook.shape[0]
    N = B * T
    f = features.reshape(N, D)
    TN = min(1024, N)
    nt = N // TN

    ids_t, sums = pl.pallas_call(
        _vq_tile,
        grid=(nt,),
        in_specs=[
            pl.BlockSpec((TN, D), lambda i: (i, 0)),
            pl.BlockSpec((C, D), lambda i: (0, 0)),
        ],
        out_specs=[
            pl.BlockSpec((1, 1, TN), lambda i: (i, 0, 0)),
            pl.BlockSpec((1, 1, 128), lambda i: (i, 0, 0)),
        ],
        out_shape=[
            jax.ShapeDtypeStruct((nt, 1, TN), jnp.int32),
            jax.ShapeDtypeStruct((nt, 1, 128), jnp.float32),
        ],
        scratch_shapes=[
            pltpu.VMEM((C,), jnp.float32),
            pltpu.VMEM((C, D), jnp.bfloat16),
            pltpu.VMEM((1, C), jnp.float32),
        ],
        compiler_params=pltpu.CompilerParams(
            dimension_semantics=("arbitrary",),
        ),
    )(f, codebook)

    ids = ids_t.reshape(B, T)
    mse = jnp.sum(sums[:, 0, 0]) / jnp.float32(N * D)
    commit = mse + 0.25 * mse
    return (ids, commit)


# speedup vs baseline: 1.2685x; 1.2685x over previous
import jax
import jax.numpy as jnp
from jax import lax
from jax.experimental import pallas as pl
from jax.experimental.pallas import tpu as pltpu


def _vq_tile(f_ref, c_ref, ids_ref, sum_ref, csq_ref, c16_ref, iota_ref):
    i = pl.program_id(0)

    @pl.when(i == 0)
    def _():
        c = c_ref[...]
        csq_ref[...] = jnp.sum(c * c, axis=1)
        c16_ref[...] = c.astype(jnp.bfloat16)
        iota_ref[...] = lax.broadcasted_iota(
            jnp.int32, iota_ref.shape, 1).astype(jnp.float32)

    f = f_ref[...]
    fsq = jnp.sum(f * f, axis=1, keepdims=True)
    b2 = lax.dot_general(
        f.astype(jnp.bfloat16) * jnp.bfloat16(2.0), c16_ref[...],
        dimension_numbers=(((1,), (1,)), ((), ())),
        preferred_element_type=jnp.float32,
    )
    d = (fsq - b2) + csq_ref[...][None, :]
    m = jnp.min(d, axis=1, keepdims=True)
    idx = jnp.min(jnp.where(d == m, iota_ref[...], jnp.float32(3.0e38)),
                  axis=1)
    ids_ref[0, 0, :] = idx.astype(jnp.int32)
    sum_ref[0, 0, :] = jnp.full((128,), jnp.sum(m), dtype=jnp.float32)


def kernel(features, codebook):
    B, T, D = features.shape
    C = codebook.shape[0]
    N = B * T
    f = features.reshape(N, D)
    TN = min(1024, N)
    nt = N // TN

    ids_t, sums = pl.pallas_call(
        _vq_tile,
        grid=(nt,),
        in_specs=[
            pl.BlockSpec((TN, D), lambda i: (i, 0)),
            pl.BlockSpec((C, D), lambda i: (0, 0)),
        ],
        out_specs=[
            pl.BlockSpec((1, 1, TN), lambda i: (i, 0, 0)),
            pl.BlockSpec((1, 1, 128), lambda i: (i, 0, 0)),
        ],
        out_shape=[
            jax.ShapeDtypeStruct((nt, 1, TN), jnp.int32),
            jax.ShapeDtypeStruct((nt, 1, 128), jnp.float32),
        ],
        scratch_shapes=[
            pltpu.VMEM((C,), jnp.float32),
            pltpu.VMEM((C, D), jnp.bfloat16),
            pltpu.VMEM((1, C), jnp.float32),
        ],
        compiler_params=pltpu.CompilerParams(
            dimension_semantics=("arbitrary",),
        ),
    )(f, codebook)

    ids = ids_t.reshape(B, T)
    mse = jnp.sum(sums[:, 0, 0]) / jnp.float32(N * D)
    commit = mse + 0.25 * mse
    return (ids, commit)
